# baseline (speedup 1.0000x reference)
import functools

import jax
import jax.numpy as jnp
from jax import lax
from jax.experimental import pallas as pl
from jax.experimental.pallas import tpu as pltpu
from jax.experimental.pallas import tpu_sc as plsc

B = 16384
D = 64
NC = 2
NS = 16
NW = NC * NS
BPW = B // NW
CHUNK = 128
NCHUNK = BPW // CHUNK
L = 16


def _rsqrt(x):
    i = plsc.bitcast(x, jnp.int32)
    i = jnp.int32(0x5F3759DF) - lax.shift_right_arithmetic(i, 1)
    y = plsc.bitcast(i, jnp.float32)
    for _ in range(3):
        y = y * (1.5 - 0.5 * x * y * y)
    return y


def _sc_kernel(heads_hbm, rels_hbm, tails_hbm, node_hbm, rel_hbm, out_hbm,
               hidx, ridx, tidx, hbuf, rbuf, tbuf, obuf, sem):
    wid = lax.axis_index("s") * NC + lax.axis_index("c")
    base = wid * BPW

    pltpu.sync_copy(heads_hbm.at[wid], hidx)
    pltpu.sync_copy(rels_hbm.at[wid], ridx)
    pltpu.sync_copy(tails_hbm.at[wid], tidx)

    copies = []
    for j in range(NCHUNK):
        dst = slice(j * CHUNK, (j + 1) * CHUNK)
        copies.append(pltpu.async_copy(node_hbm.at[hidx.at[j]],
                                       hbuf.at[dst, :], sem))
        copies.append(pltpu.async_copy(rel_hbm.at[ridx.at[j]],
                                       rbuf.at[dst, :], sem))
        copies.append(pltpu.async_copy(node_hbm.at[tidx.at[j]],
                                       tbuf.at[dst, :], sem))
    for c in copies:
        c.wait()

    lanes = lax.iota(jnp.int32, L)
    zero = jnp.zeros((L,), jnp.float32)

    def group(g, _):
        rows = g * L + lanes
        nh = zero; nt = zero; nr = zero
        hr = zero; ht = zero; rt = zero
        for d in range(D):
            dcol = jnp.full((L,), d, jnp.int32)
            gh = plsc.load_gather(hbuf, [rows, dcol])
            gr = plsc.load_gather(rbuf, [rows, dcol])
            gt = plsc.load_gather(tbuf, [rows, dcol])
            nh = nh + gh * gh
            nt = nt + gt * gt
            nr = nr + gr * gr
            hr = hr + gh * gr
            ht = ht + gh * gt
            rt = rt + gr * gt
        ih = _rsqrt(jnp.maximum(nh, 1e-24))
        it = _rsqrt(jnp.maximum(nt, 1e-24))
        s = (nh * ih * ih + nr + nt * it * it
             + 2.0 * hr * ih - 2.0 * (ht * ih) * it - 2.0 * rt * it)
        s = jnp.maximum(s, 0.0)
        score = -(s * _rsqrt(jnp.maximum(s, 1e-30)))
        plsc.store_scatter(obuf, [rows], score)
        return _

    lax.fori_loop(0, BPW // L, group, None)
    pltpu.sync_copy(obuf, out_hbm.at[pl.ds(base, BPW)])


@jax.jit
def kernel(batched_paths, node_emb, rel_emb):
    heads = batched_paths[:, 2].reshape(NW, NCHUNK, CHUNK)
    rels = batched_paths[:, 1].reshape(NW, NCHUNK, CHUNK)
    tails = batched_paths[:, 0].reshape(NW, NCHUNK, CHUNK)

    mesh = plsc.VectorSubcoreMesh(core_axis_name="c", subcore_axis_name="s",
                                  num_cores=NC, num_subcores=NS)
    run = pl.kernel(
        _sc_kernel,
        out_type=jax.ShapeDtypeStruct((B,), jnp.float32),
        mesh=mesh,
        compiler_params=pltpu.CompilerParams(needs_layout_passes=False, use_tc_tiling_on_sc=False),
        scratch_types=[
            pltpu.VMEM((NCHUNK, CHUNK), jnp.int32),
            pltpu.VMEM((NCHUNK, CHUNK), jnp.int32),
            pltpu.VMEM((NCHUNK, CHUNK), jnp.int32),
            pltpu.VMEM((BPW, D), jnp.float32),
            pltpu.VMEM((BPW, D), jnp.float32),
            pltpu.VMEM((BPW, D), jnp.float32),
            pltpu.VMEM((BPW,), jnp.float32),
            pltpu.SemaphoreType.DMA,
        ],
    )
    return run(heads, rels, tails, node_emb, rel_emb)
